# baseline (speedup 1.0000x reference)
import functools

import jax
import jax.numpy as jnp
from jax import lax
from jax.experimental import pallas as pl
from jax.experimental.pallas import tpu as pltpu
from jax.experimental.pallas import tpu_sc as plsc

V = 65
VP = 80
B, T = 64, 2048
N = B * T

NC, NS, L = 2, 16, 16
NW = NC * NS
BG, TG = 8, 4
BB, TB = B // BG, T // TG
VC = 13
NSL = V // VC


def _lse_body(table_ref, lse_ref):
    x = table_ref[...]
    m = jnp.max(x, axis=-1)
    lse_ref[...] = m + jnp.log(jnp.sum(jnp.exp(x - m[:, None]), axis=-1))


_lse = pl.pallas_call(
    _lse_body,
    out_shape=jax.ShapeDtypeStruct((V,), jnp.float32),
)


_sc_mesh = plsc.VectorSubcoreMesh(
    core_axis_name="c", subcore_axis_name="s", num_cores=NC, num_subcores=NS
)


@functools.partial(
    pl.kernel,
    out_type=(
        jax.ShapeDtypeStruct((V, B, T), jnp.float32),
        jax.ShapeDtypeStruct((NW, L), jnp.float32),
        jax.ShapeDtypeStruct((NW * VP,), jnp.float32),
    ),
    mesh=_sc_mesh,
    compiler_params=pltpu.CompilerParams(
        needs_layout_passes=False, use_tc_tiling_on_sc=True,
        disable_bounds_checks=True,
    ),
    scratch_types=[
        pltpu.VMEM((BB, TB), jnp.int32),
        pltpu.VMEM((BB, TB), jnp.int32),
        pltpu.VMEM((VC, BB, TB), jnp.float32),
        pltpu.VMEM((VC, BB, TB), jnp.float32),
        pltpu.VMEM((V * V,), jnp.float32),
        pltpu.VMEM((VP,), jnp.float32),
        pltpu.VMEM((L,), jnp.float32),
        pltpu.SemaphoreType.DMA,
        pltpu.SemaphoreType.DMA,
        pltpu.SemaphoreType.DMA,
    ],
)
def _sc_body(inp_hbm, tgt_hbm, tab_hbm, out_hbm, part_hbm, hist_hbm,
             idx_v, tgt_v, slab_a, slab_b, tab_v, hist_v, part_v,
             sem_a, sem_b, sem_c):
    wid = lax.axis_index("s") * NC + lax.axis_index("c")
    bg = wid // TG
    tg = wid - bg * TG
    b0 = bg * BB
    t0 = tg * TB

    tab_dma = pltpu.async_copy(tab_hbm, tab_v, sem_a)
    idx_dma = pltpu.async_copy(
        inp_hbm.at[pl.ds(b0, BB), pl.ds(t0, TB)], idx_v, sem_b
    )
    tgt_dma = pltpu.async_copy(
        tgt_hbm.at[pl.ds(b0, BB), pl.ds(t0, TB)], tgt_v, sem_c
    )
    tab_dma.wait()
    idx_dma.wait()

    bufs = (slab_a, slab_b)
    sems = (sem_a, sem_b)
    pending = [None, None]
    for s in range(NSL):
        slot = s % 2
        buf = bufs[slot]
        if pending[slot] is not None:
            pending[slot].wait()

        def row(bq, carry, _s=s, _buf=buf):
            def build(tq):
                iv = idx_v[bq, pl.ds(tq * L, L)]
                for vi in range(VC):
                    _buf[vi, bq, pl.ds(tq * L, L)] = plsc.load_gather(
                        tab_v, [iv * V + (_s * VC + vi)]
                    )

            plsc.parallel_loop(0, TB // L, unroll=8)(build)
            return carry

        lax.fori_loop(0, BB, row, 0)
        pending[slot] = pltpu.async_copy(
            buf,
            out_hbm.at[pl.ds(s * VC, VC), pl.ds(b0, BB), pl.ds(t0, TB)],
            sems[slot],
        )

    tgt_dma.wait()
    for k in range(VP // L):
        hist_v[pl.ds(k * L, L)] = jnp.zeros((L,), jnp.float32)
    ones = jnp.ones((L,), jnp.float32)
    total = jnp.zeros((L,), jnp.float32)
    for bq in range(BB):
        def loss_group(tq, acc, _bq=bq):
            iv = idx_v[_bq, pl.ds(tq * L, L)]
            tv = tgt_v[_bq, pl.ds(tq * L, L)]
            plsc.addupdate_scatter(hist_v, [iv], ones)
            return acc + plsc.load_gather(tab_v, [iv * V + tv])

        total = lax.fori_loop(0, TB // L, loss_group, total)
    part_v[...] = total * (1.0 / N)
    pltpu.sync_copy(part_v, part_hbm.at[wid])
    pltpu.sync_copy(hist_v, hist_hbm.at[pl.ds(wid * VP, VP)])

    pending[0].wait()
    pending[1].wait()


def kernel(input, target, table):
    lse = _lse(table)
    logits_vbt, parts, hists = _sc_body(input, target, table.reshape(V * V))
    counts = hists.reshape(NW, VP)[:, :V].sum(axis=0)
    loss = jnp.dot(counts, lse) * (1.0 / N) - jnp.sum(parts)
    return jnp.transpose(logits_vbt, (1, 2, 0)), loss


# speedup vs baseline: 1.0049x; 1.0049x over previous
import functools

import jax
import jax.numpy as jnp
from jax import lax
from jax.experimental import pallas as pl
from jax.experimental.pallas import tpu as pltpu
from jax.experimental.pallas import tpu_sc as plsc

V = 65
VP = 96
B, T = 64, 2048
N = B * T

NC, NS, L = 2, 16, 16
NW = NC * NS
BG, TG = 8, 4
BB, TB = B // BG, T // TG
VC = 13
NSL = V // VC


def _lse_body(table_ref, lse_ref):
    x = table_ref[...]
    m = jnp.max(x, axis=-1)
    lse_ref[...] = m + jnp.log(jnp.sum(jnp.exp(x - m[:, None]), axis=-1))


_lse = pl.pallas_call(
    _lse_body,
    out_shape=jax.ShapeDtypeStruct((V,), jnp.float32),
)


_sc_mesh = plsc.VectorSubcoreMesh(
    core_axis_name="c", subcore_axis_name="s", num_cores=NC, num_subcores=NS
)


@functools.partial(
    pl.kernel,
    out_type=(
        jax.ShapeDtypeStruct((V, B, T), jnp.float32),
        jax.ShapeDtypeStruct((NW, VP), jnp.float32),
    ),
    mesh=_sc_mesh,
    compiler_params=pltpu.CompilerParams(
        needs_layout_passes=False, use_tc_tiling_on_sc=True,
        disable_bounds_checks=True,
    ),
    scratch_types=[
        pltpu.VMEM((BB, TB), jnp.int32),
        pltpu.VMEM((BB, TB), jnp.int32),
        pltpu.VMEM((VC, BB, TB), jnp.float32),
        pltpu.VMEM((VC, BB, TB), jnp.float32),
        pltpu.VMEM((V * V,), jnp.float32),
        pltpu.VMEM((VP,), jnp.float32),
        pltpu.SemaphoreType.DMA,
        pltpu.SemaphoreType.DMA,
        pltpu.SemaphoreType.DMA,
    ],
)
def _sc_body(inp_hbm, tgt_hbm, tab_hbm, out_hbm, hist_hbm,
             idx_v, tgt_v, slab_a, slab_b, tab_v, hist_v,
             sem_a, sem_b, sem_c):
    wid = lax.axis_index("s") * NC + lax.axis_index("c")
    bg = wid // TG
    tg = wid - bg * TG
    b0 = bg * BB
    t0 = tg * TB

    tab_dma = pltpu.async_copy(tab_hbm, tab_v, sem_a)
    idx_dma = pltpu.async_copy(
        inp_hbm.at[pl.ds(b0, BB), pl.ds(t0, TB)], idx_v, sem_b
    )
    tgt_dma = pltpu.async_copy(
        tgt_hbm.at[pl.ds(b0, BB), pl.ds(t0, TB)], tgt_v, sem_c
    )
    tab_dma.wait()
    idx_dma.wait()

    bufs = (slab_a, slab_b)
    sems = (sem_a, sem_b)
    pending = [None, None]
    for s in range(NSL):
        slot = s % 2
        buf = bufs[slot]
        if pending[slot] is not None:
            pending[slot].wait()

        def row(bq, carry, _s=s, _buf=buf):
            def build(tq):
                iv = idx_v[bq, pl.ds(tq * L, L)]
                for vi in range(VC):
                    _buf[vi, bq, pl.ds(tq * L, L)] = plsc.load_gather(
                        tab_v, [iv * V + (_s * VC + vi)]
                    )

            plsc.parallel_loop(0, TB // L, unroll=8)(build)
            return carry

        lax.fori_loop(0, BB, row, 0)
        pending[slot] = pltpu.async_copy(
            buf,
            out_hbm.at[pl.ds(s * VC, VC), pl.ds(b0, BB), pl.ds(t0, TB)],
            sems[slot],
        )

    tgt_dma.wait()
    for k in range(VP // L):
        hist_v[pl.ds(k * L, L)] = jnp.zeros((L,), jnp.float32)
    ones = jnp.ones((L,), jnp.float32)
    total = jnp.zeros((L,), jnp.float32)
    for bq in range(BB):
        def loss_group(tq, acc, _bq=bq):
            iv = idx_v[_bq, pl.ds(tq * L, L)]
            tv = tgt_v[_bq, pl.ds(tq * L, L)]
            plsc.addupdate_scatter(hist_v, [iv], ones)
            return acc + plsc.load_gather(tab_v, [iv * V + tv])

        total = lax.fori_loop(0, TB // L, loss_group, total)
    hist_v[pl.ds(VP - L, L)] = total * (1.0 / N)
    pltpu.sync_copy(hist_v, hist_hbm.at[wid])

    pending[0].wait()
    pending[1].wait()


def kernel(input, target, table):
    lse = _lse(table)
    logits_vbt, hists = _sc_body(input, target, table.reshape(V * V))
    counts = hists[:, :V].sum(axis=0)
    loss = jnp.dot(counts, lse) * (1.0 / N) - jnp.sum(hists[:, VP - L:])
    return jnp.transpose(logits_vbt, (1, 2, 0)), loss


# speedup vs baseline: 1.0859x; 1.0806x over previous
import functools

import jax
import jax.numpy as jnp
from jax import lax
from jax.experimental import pallas as pl
from jax.experimental.pallas import tpu as pltpu
from jax.experimental.pallas import tpu_sc as plsc

V = 65
B, T = 64, 2048
N = B * T

NC, NS, L = 2, 16, 16
NW = NC * NS
BG, TG = 8, 4
BB, TB = B // BG, T // TG
VC = 13
NSL = V // VC


def _prep_body(table_ref, nll_ref, tabt_ref):
    x = table_ref[...]
    m = jnp.max(x, axis=-1, keepdims=True)
    lse = m + jnp.log(jnp.sum(jnp.exp(x - m), axis=-1, keepdims=True))
    nll_ref[...] = lse - x
    tabt_ref[...] = x.T


_prep = pl.pallas_call(
    _prep_body,
    out_shape=(
        jax.ShapeDtypeStruct((V, V), jnp.float32),
        jax.ShapeDtypeStruct((V, V), jnp.float32),
    ),
)


_sc_mesh = plsc.VectorSubcoreMesh(
    core_axis_name="c", subcore_axis_name="s", num_cores=NC, num_subcores=NS
)


@functools.partial(
    pl.kernel,
    out_type=(
        jax.ShapeDtypeStruct((V, B, T), jnp.float32),
        jax.ShapeDtypeStruct((NW, L), jnp.float32),
    ),
    mesh=_sc_mesh,
    compiler_params=pltpu.CompilerParams(
        needs_layout_passes=False, use_tc_tiling_on_sc=True,
        disable_bounds_checks=True,
    ),
    scratch_types=[
        pltpu.VMEM((BB, TB), jnp.int32),
        pltpu.VMEM((BB, TB), jnp.int32),
        pltpu.VMEM((VC, BB, TB), jnp.float32),
        pltpu.VMEM((VC, BB, TB), jnp.float32),
        pltpu.VMEM((V * V,), jnp.float32),
        pltpu.VMEM((V * V,), jnp.float32),
        pltpu.VMEM((L,), jnp.float32),
        pltpu.SemaphoreType.DMA,
        pltpu.SemaphoreType.DMA,
        pltpu.SemaphoreType.DMA,
        pltpu.SemaphoreType.DMA,
    ],
)
def _sc_body(inp_hbm, tgt_hbm, tabt_hbm, nll_hbm, out_hbm, part_hbm,
             idx_v, tgt_v, slab_a, slab_b, tabt_v, nll_v, part_v,
             sem_a, sem_b, sem_c, sem_d):
    wid = lax.axis_index("s") * NC + lax.axis_index("c")
    bg = wid // TG
    tg = wid - bg * TG
    b0 = bg * BB
    t0 = tg * TB

    tab_dma = pltpu.async_copy(tabt_hbm, tabt_v, sem_a)
    idx_dma = pltpu.async_copy(
        inp_hbm.at[pl.ds(b0, BB), pl.ds(t0, TB)], idx_v, sem_b
    )
    tgt_dma = pltpu.async_copy(
        tgt_hbm.at[pl.ds(b0, BB), pl.ds(t0, TB)], tgt_v, sem_c
    )
    nll_dma = pltpu.async_copy(nll_hbm, nll_v, sem_d)
    tab_dma.wait()
    idx_dma.wait()

    bufs = (slab_a, slab_b)
    sems = (sem_a, sem_b)
    pending = [None, None]
    for s in range(NSL):
        slot = s % 2
        buf = bufs[slot]
        if pending[slot] is not None:
            pending[slot].wait()

        def row(bq, carry, _s=s, _buf=buf):
            def build(tq):
                iv = idx_v[bq, pl.ds(tq * L, L)]
                for vi in range(VC):
                    _buf[vi, bq, pl.ds(tq * L, L)] = plsc.load_gather(
                        tabt_v, [iv + (_s * VC + vi) * V]
                    )

            plsc.parallel_loop(0, TB // L, unroll=16)(build)
            return carry

        lax.fori_loop(0, BB, row, 0)
        pending[slot] = pltpu.async_copy(
            buf,
            out_hbm.at[pl.ds(s * VC, VC), pl.ds(b0, BB), pl.ds(t0, TB)],
            sems[slot],
        )

    tgt_dma.wait()
    nll_dma.wait()
    total = jnp.zeros((L,), jnp.float32)
    for bq in range(BB):
        def loss_group(tq, acc, _bq=bq):
            iv = idx_v[_bq, pl.ds(tq * L, L)]
            tv = tgt_v[_bq, pl.ds(tq * L, L)]
            return acc + plsc.load_gather(nll_v, [iv * V + tv])

        total = lax.fori_loop(0, TB // L, loss_group, total)
    part_v[...] = total * (1.0 / N)
    pltpu.sync_copy(part_v, part_hbm.at[wid])

    pending[0].wait()
    pending[1].wait()


def kernel(input, target, table):
    nll, tabt = _prep(table)
    logits_vbt, parts = _sc_body(
        input, target, tabt.reshape(V * V), nll.reshape(V * V)
    )
    return jnp.transpose(logits_vbt, (1, 2, 0)), jnp.sum(parts)
